# baseline (speedup 1.0000x reference)
import functools

import jax
import jax.numpy as jnp
import numpy as np
from jax import lax
from jax.experimental import pallas as pl
from jax.experimental.pallas import tpu as pltpu
from jax.experimental.pallas import tpu_sc as plsc

_IN = 64
_OUT = 96
_C = 8
_L = 16
_ROWW = _IN * _C
_OROWW = _OUT * _C
_NROWS = 18
_QY = 24
_NW = 32
_NTASK = 2 * _OUT * 4
_PER_W = _NTASK // _NW


def _interp_1d():
    loc = np.linspace(0.0, _IN - 1.0, _OUT)
    k0 = np.clip(np.floor(loc), 0, _IN - 1).astype(np.int64)
    k1 = np.clip(k0 + 1, 0, _IN - 1)
    w0 = k1.astype(np.float64) - loc
    return k0, k1, w0.astype(np.float32)


def _tables():
    k0, k1, w0 = _interp_1d()
    w0rep = np.repeat(w0[:, None], _L, axis=1).astype(np.float32)
    p = np.arange(_OUT * _C)
    jx, c = p // _C, p % _C
    idx0 = (k0[jx] * _C + c).astype(np.int32).reshape(-1, _L)
    idx1 = (k1[jx] * _C + c).astype(np.int32).reshape(-1, _L)
    b0 = w0[jx].astype(np.float32).reshape(-1, _L)
    return w0rep, idx0, idx1, b0


def _body(x_hbm, w0_hbm, idx0_hbm, idx1_hbm, b0_hbm, out_hbm,
          in0_v, in1_v, zl_v, ty_v, ob_v, w0_v, idx0_v, idx1_v, b0_v):
    wid = lax.axis_index("s") * 2 + lax.axis_index("c")

    pltpu.sync_copy(w0_hbm, w0_v)
    pltpu.sync_copy(idx0_hbm, idx0_v)
    pltpu.sync_copy(idx1_hbm, idx1_v)
    pltpu.sync_copy(b0_hbm, b0_v)

    one = jnp.float32(1.0)

    def task(i, _):
        t = wid * _PER_W + i
        b = jnp.where(t >= _NTASK // 2, 1, 0)
        rem = t - b * (_NTASK // 2)
        jz = rem >> 2
        q = rem & 3
        kz0 = (jz * (63 * 690)) >> 16
        kz1 = jnp.minimum(kz0 + 1, _IN - 1)
        lo = jnp.minimum((q * (24 * 63 * 690)) >> 16, _IN - _NROWS)

        pltpu.sync_copy(x_hbm.at[b, kz0, pl.ds(lo, _NROWS)], in0_v)
        pltpu.sync_copy(x_hbm.at[b, kz1, pl.ds(lo, _NROWS)], in1_v)

        wz0 = w0_v[jz]
        wz1 = one - wz0

        def zlerp(r, _):
            row = r >> 5
            col = (r & 31) * _L
            zl_v[row, pl.ds(col, _L)] = (
                wz0 * in0_v[row, pl.ds(col, _L)] + wz1 * in1_v[row, pl.ds(col, _L)]
            )
            return 0

        lax.fori_loop(0, _NROWS * (_ROWW // _L), zlerp, 0)

        def yrow(jl, _):
            jy = q * _QY + jl
            ky0 = (jy * (63 * 690)) >> 16
            ky1 = jnp.minimum(ky0 + 1, _IN - 1)
            r0 = ky0 - lo
            r1 = ky1 - lo
            wy0 = w0_v[jy]
            wy1 = one - wy0

            def ylerp(u, _):
                col = u * _L
                ty_v[pl.ds(col, _L)] = (
                    wy0 * zl_v[r0, pl.ds(col, _L)] + wy1 * zl_v[r1, pl.ds(col, _L)]
                )
                return 0

            lax.fori_loop(0, _ROWW // _L, ylerp, 0)

            def xlerp(v, _):
                g0 = plsc.load_gather(ty_v, [idx0_v[v]])
                g1 = plsc.load_gather(ty_v, [idx1_v[v]])
                bw0 = b0_v[v]
                ob_v[jl, pl.ds(v * _L, _L)] = bw0 * g0 + (one - bw0) * g1
                return 0

            lax.fori_loop(0, _OROWW // _L, xlerp, 0)
            return 0

        lax.fori_loop(0, _QY, yrow, 0)

        pltpu.sync_copy(ob_v, out_hbm.at[b, jz, pl.ds(q * _QY, _QY)])
        return 0

    lax.fori_loop(0, _PER_W, task, 0)


@jax.jit
def kernel(x):
    B = x.shape[0]
    w0rep, idx0, idx1, b0 = _tables()
    mesh = plsc.VectorSubcoreMesh(
        core_axis_name="c", subcore_axis_name="s", num_cores=2, num_subcores=16
    )
    run = functools.partial(
        pl.kernel,
        out_type=jax.ShapeDtypeStruct((B, _OUT, _OUT, _OROWW), jnp.float32),
        mesh=mesh,
        compiler_params=pltpu.CompilerParams(use_tc_tiling_on_sc=False, needs_layout_passes=False),
        scratch_types=[
            pltpu.VMEM((_NROWS, _ROWW), jnp.float32),
            pltpu.VMEM((_NROWS, _ROWW), jnp.float32),
            pltpu.VMEM((_NROWS, _ROWW), jnp.float32),
            pltpu.VMEM((_ROWW,), jnp.float32),
            pltpu.VMEM((_QY, _OROWW), jnp.float32),
            pltpu.VMEM((_OUT, _L), jnp.float32),
            pltpu.VMEM((_OROWW // _L, _L), jnp.int32),
            pltpu.VMEM((_OROWW // _L, _L), jnp.int32),
            pltpu.VMEM((_OROWW // _L, _L), jnp.float32),
        ],
    )(_body)
    x2 = x.reshape(B, _IN, _IN, _ROWW)
    out = run(x2, jnp.asarray(w0rep), jnp.asarray(idx0), jnp.asarray(idx1),
              jnp.asarray(b0))
    return out.reshape(B, _OUT, _OUT, _OUT, _C)


# speedup vs baseline: 1.5073x; 1.5073x over previous
import functools

import jax
import jax.numpy as jnp
import numpy as np
from jax import lax
from jax.experimental import pallas as pl
from jax.experimental.pallas import tpu as pltpu
from jax.experimental.pallas import tpu_sc as plsc

_IN = 64
_OUT = 96
_C = 8
_L = 16
_ROWW = _IN * _C
_OROWW = _OUT * _C
_QY = 24
_NW = 32
_NTASK = 2 * _OUT
_PER_W = _NTASK // _NW
_RVR = _ROWW // _L
_OVR = _OROWW // _L


def _interp_1d():
    loc = np.linspace(0.0, _IN - 1.0, _OUT)
    k0 = np.clip(np.floor(loc), 0, _IN - 1).astype(np.int64)
    k1 = np.clip(k0 + 1, 0, _IN - 1)
    w0 = k1.astype(np.float64) - loc
    return k0, k1, w0.astype(np.float32)


def _tables():
    k0, k1, w0 = _interp_1d()
    w0rep = np.repeat(w0, _L).astype(np.float32)
    p = np.arange(_OUT * _C)
    jx, c = p // _C, p % _C
    idx0 = (k0[jx] * _C + c).astype(np.int32)
    idx1 = (k1[jx] * _C + c).astype(np.int32)
    b0 = w0[jx].astype(np.float32)
    return w0rep, idx0, idx1, b0


def _body(x_hbm, w0_hbm, idx0_hbm, idx1_hbm, b0_hbm, out_hbm,
          pa_v, pb_v, tyq_v, ob0_v, ob1_v, w0_v, idx0_v, idx1_v, b0_v,
          sem_ina, sem_inb, sem_o0, sem_o1):
    wid = lax.axis_index("s") * 2 + lax.axis_index("c")

    pltpu.sync_copy(w0_hbm, w0_v)
    pltpu.sync_copy(idx0_hbm, idx0_v)
    pltpu.sync_copy(idx1_hbm, idx1_v)
    pltpu.sync_copy(b0_hbm, b0_v)

    one = jnp.float32(1.0)
    obufs = (ob0_v, ob1_v)
    osems = (sem_o0, sem_o1)

    def task(i, _):
        t = wid * _PER_W + i
        b = jnp.where(t >= _OUT, 1, 0)
        jz = t - b * _OUT
        kz0 = (jz * (63 * 690)) >> 16
        kz1 = jnp.minimum(kz0 + 1, _IN - 1)

        ha = pltpu.async_copy(x_hbm.at[b, kz0], pa_v, sem_ina)
        hb = pltpu.async_copy(x_hbm.at[b, kz1], pb_v, sem_inb)
        ha.wait()
        hb.wait()

        wz0 = w0_v[pl.ds(jz * _L, _L)]
        wz1 = one - wz0

        def zlerp(rr, _):
            for u in range(_RVR):
                s = pl.ds(u * _L, _L)
                pa_v[rr, s] = wz0 * pa_v[rr, s] + wz1 * pb_v[rr, s]
            return 0

        lax.fori_loop(0, _IN, zlerp, 0)

        out_handles = [None, None]
        for Q in range(4):
            ob_v = obufs[Q % 2]
            if out_handles[Q % 2] is not None:
                out_handles[Q % 2].wait()

            def yrow(jl, _):
                jy = Q * _QY + jl
                ky0 = (jy * (63 * 690)) >> 16
                ky1 = jnp.minimum(ky0 + 1, _IN - 1)
                wy0 = w0_v[pl.ds(jy * _L, _L)]
                wy1 = one - wy0
                base = jl * _ROWW
                for u in range(_RVR):
                    s = pl.ds(u * _L, _L)
                    tyq_v[pl.ds(base + u * _L, _L)] = (
                        wy0 * pa_v[ky0, s] + wy1 * pa_v[ky1, s]
                    )
                return 0

            lax.fori_loop(0, _QY, yrow, 0)

            for v in range(_OVR):
                iv0 = idx0_v[pl.ds(v * _L, _L)]
                iv1 = idx1_v[pl.ds(v * _L, _L)]
                bw0 = b0_v[pl.ds(v * _L, _L)]
                bw1 = one - bw0

                def xrow(jl, _):
                    off = jl * _ROWW
                    g0 = plsc.load_gather(tyq_v, [iv0 + off])
                    g1 = plsc.load_gather(tyq_v, [iv1 + off])
                    ob_v[jl, pl.ds(v * _L, _L)] = bw0 * g0 + bw1 * g1
                    return 0

                lax.fori_loop(0, _QY, xrow, 0)

            out_handles[Q % 2] = pltpu.async_copy(
                ob_v, out_hbm.at[b, jz, pl.ds(Q * _QY, _QY)], osems[Q % 2]
            )

        out_handles[0].wait()
        out_handles[1].wait()
        return 0

    lax.fori_loop(0, _PER_W, task, 0)


@jax.jit
def kernel(x):
    B = x.shape[0]
    w0rep, idx0, idx1, b0 = _tables()
    mesh = plsc.VectorSubcoreMesh(
        core_axis_name="c", subcore_axis_name="s", num_cores=2, num_subcores=16
    )
    run = functools.partial(
        pl.kernel,
        out_type=jax.ShapeDtypeStruct((B, _OUT, _OUT, _OROWW), jnp.float32),
        mesh=mesh,
        compiler_params=pltpu.CompilerParams(needs_layout_passes=False),
        scratch_types=[
            pltpu.VMEM((_IN, _ROWW), jnp.float32),
            pltpu.VMEM((_IN, _ROWW), jnp.float32),
            pltpu.VMEM((_QY * _ROWW,), jnp.float32),
            pltpu.VMEM((_QY, _OROWW), jnp.float32),
            pltpu.VMEM((_QY, _OROWW), jnp.float32),
            pltpu.VMEM((_OUT * _L,), jnp.float32),
            pltpu.VMEM((_OROWW,), jnp.int32),
            pltpu.VMEM((_OROWW,), jnp.int32),
            pltpu.VMEM((_OROWW,), jnp.float32),
            pltpu.SemaphoreType.DMA,
            pltpu.SemaphoreType.DMA,
            pltpu.SemaphoreType.DMA,
            pltpu.SemaphoreType.DMA,
        ],
    )(_body)
    x2 = x.reshape(B, _IN, _IN, _ROWW)
    out = run(x2, jnp.asarray(w0rep), jnp.asarray(idx0), jnp.asarray(idx1),
              jnp.asarray(b0))
    return out.reshape(B, _OUT, _OUT, _OUT, _C)


# speedup vs baseline: 1.5254x; 1.0120x over previous
import functools

import jax
import jax.numpy as jnp
import numpy as np
from jax import lax
from jax.experimental import pallas as pl
from jax.experimental.pallas import tpu as pltpu
from jax.experimental.pallas import tpu_sc as plsc

_IN = 64
_OUT = 96
_C = 8
_L = 16
_ROWW = _IN * _C
_OROWW = _OUT * _C
_QY = 24
_NW = 32
_NTASK = 2 * _OUT
_PER_W = _NTASK // _NW
_RVR = _ROWW // _L
_OVR = _OROWW // _L


def _interp_1d():
    loc = np.linspace(0.0, _IN - 1.0, _OUT)
    k0 = np.clip(np.floor(loc), 0, _IN - 1).astype(np.int64)
    k1 = np.clip(k0 + 1, 0, _IN - 1)
    w0 = k1.astype(np.float64) - loc
    return k0, k1, w0.astype(np.float32)


def _tables():
    k0, k1, w0 = _interp_1d()
    w0rep = np.repeat(w0, _L).astype(np.float32)
    p = np.arange(_OUT * _C)
    jx, c = p // _C, p % _C
    idx0 = (k0[jx] * _C + c).astype(np.int32)
    idx1 = (k1[jx] * _C + c).astype(np.int32)
    b0 = w0[jx].astype(np.float32)
    return w0rep, idx0, idx1, b0


def _body(x_hbm, w0_hbm, idx0_hbm, idx1_hbm, b0_hbm, out_hbm,
          pa_v, pb_v, tyq_v, ob0_v, ob1_v, w0_v, idx0_v, idx1_v, b0_v,
          sem_ina, sem_inb, sem_o0, sem_o1):
    wid = lax.axis_index("s") * 2 + lax.axis_index("c")

    pltpu.sync_copy(w0_hbm, w0_v)
    pltpu.sync_copy(idx0_hbm, idx0_v)
    pltpu.sync_copy(idx1_hbm, idx1_v)
    pltpu.sync_copy(b0_hbm, b0_v)

    one = jnp.float32(1.0)
    obufs = (ob0_v, ob1_v)
    osems = (sem_o0, sem_o1)

    def task(i, _):
        t = wid * _PER_W + i
        b = jnp.where(t >= _OUT, 1, 0)
        jz = t - b * _OUT
        kz0 = (jz * (63 * 690)) >> 16
        kz1 = jnp.minimum(kz0 + 1, _IN - 1)

        ha = pltpu.async_copy(x_hbm.at[b, kz0], pa_v, sem_ina)
        hb = pltpu.async_copy(x_hbm.at[b, kz1], pb_v, sem_inb)
        ha.wait()
        hb.wait()

        wz0 = w0_v[pl.ds(jz * _L, _L)]
        wz1 = one - wz0

        def zlerp(rr, _):
            for u in range(_RVR):
                s = pl.ds(u * _L, _L)
                pa_v[rr, s] = wz0 * pa_v[rr, s] + wz1 * pb_v[rr, s]
            return 0

        lax.fori_loop(0, _IN, zlerp, 0)

        out_handles = [None, None]
        for Q in range(4):
            ob_v = obufs[Q % 2]
            if out_handles[Q % 2] is not None:
                out_handles[Q % 2].wait()

            def yrow(jl, _):
                jy = Q * _QY + jl
                ky0 = (jy * (63 * 690)) >> 16
                ky1 = jnp.minimum(ky0 + 1, _IN - 1)
                wy0 = w0_v[pl.ds(jy * _L, _L)]
                wy1 = one - wy0
                base = jl * _ROWW
                for u in range(_RVR):
                    s = pl.ds(u * _L, _L)
                    tyq_v[pl.ds(base + u * _L, _L)] = (
                        wy0 * pa_v[ky0, s] + wy1 * pa_v[ky1, s]
                    )
                return 0

            lax.fori_loop(0, _QY, yrow, 0)

            def xcol(v, _):
                s = pl.ds(v * _L, _L)
                iv0 = idx0_v[s]
                iv1 = idx1_v[s]
                bw0 = b0_v[s]
                bw1 = one - bw0
                for jl in range(_QY):
                    off = jl * _ROWW
                    g0 = plsc.load_gather(tyq_v, [iv0 + off])
                    g1 = plsc.load_gather(tyq_v, [iv1 + off])
                    ob_v[jl, s] = bw0 * g0 + bw1 * g1
                return 0

            lax.fori_loop(0, _OVR, xcol, 0)

            out_handles[Q % 2] = pltpu.async_copy(
                ob_v, out_hbm.at[b, jz, pl.ds(Q * _QY, _QY)], osems[Q % 2]
            )

        out_handles[0].wait()
        out_handles[1].wait()
        return 0

    lax.fori_loop(0, _PER_W, task, 0)


@jax.jit
def kernel(x):
    B = x.shape[0]
    w0rep, idx0, idx1, b0 = _tables()
    mesh = plsc.VectorSubcoreMesh(
        core_axis_name="c", subcore_axis_name="s", num_cores=2, num_subcores=16
    )
    run = functools.partial(
        pl.kernel,
        out_type=jax.ShapeDtypeStruct((B, _OUT, _OUT, _OROWW), jnp.float32),
        mesh=mesh,
        compiler_params=pltpu.CompilerParams(needs_layout_passes=False),
        scratch_types=[
            pltpu.VMEM((_IN, _ROWW), jnp.float32),
            pltpu.VMEM((_IN, _ROWW), jnp.float32),
            pltpu.VMEM((_QY * _ROWW,), jnp.float32),
            pltpu.VMEM((_QY, _OROWW), jnp.float32),
            pltpu.VMEM((_QY, _OROWW), jnp.float32),
            pltpu.VMEM((_OUT * _L,), jnp.float32),
            pltpu.VMEM((_OROWW,), jnp.int32),
            pltpu.VMEM((_OROWW,), jnp.int32),
            pltpu.VMEM((_OROWW,), jnp.float32),
            pltpu.SemaphoreType.DMA,
            pltpu.SemaphoreType.DMA,
            pltpu.SemaphoreType.DMA,
            pltpu.SemaphoreType.DMA,
        ],
    )(_body)
    x2 = x.reshape(B, _IN, _IN, _ROWW)
    out = run(x2, jnp.asarray(w0rep), jnp.asarray(idx0), jnp.asarray(idx1),
              jnp.asarray(b0))
    return out.reshape(B, _OUT, _OUT, _OUT, _C)


# speedup vs baseline: 2.5425x; 1.6668x over previous
import functools

import jax
import jax.numpy as jnp
import numpy as np
from jax import lax
from jax.experimental import pallas as pl
from jax.experimental.pallas import tpu as pltpu
from jax.experimental.pallas import tpu_sc as plsc

_IN = 64
_OUT = 96
_C = 8
_L = 16
_ROWW = _IN * _C
_OROWW = _OUT * _C
_QY = 24
_NW = 32
_NTASK = 2 * _OUT
_PER_W = _NTASK // _NW
_RVR = _ROWW // _L
_OVR = _OROWW // _L


def _interp_1d():
    loc = np.linspace(0.0, _IN - 1.0, _OUT)
    k0 = np.clip(np.floor(loc), 0, _IN - 1).astype(np.int64)
    k1 = np.clip(k0 + 1, 0, _IN - 1)
    w0 = k1.astype(np.float64) - loc
    return k0, k1, w0.astype(np.float32)


def _tables():
    k0, k1, w0 = _interp_1d()
    w0rep = np.repeat(w0, _L).astype(np.float32)
    p = np.arange(_OUT * _C)
    jx, c = p // _C, p % _C
    idx0 = (k0[jx] * _C + c).astype(np.int32)
    idx1 = (k1[jx] * _C + c).astype(np.int32)
    b0 = w0[jx].astype(np.float32)
    return w0rep, idx0, idx1, b0


def _body(x_hbm, w0_hbm, idx0_hbm, idx1_hbm, b0_hbm, out_hbm,
          pa_v, pb_v, tyq_v, ob0_v, ob1_v, w0_v, idx0_v, idx1_v, b0_v,
          sem_ina, sem_inb, sem_o0, sem_o1):
    wid = lax.axis_index("s") * 2 + lax.axis_index("c")

    pltpu.sync_copy(w0_hbm, w0_v)
    pltpu.sync_copy(idx0_hbm, idx0_v)
    pltpu.sync_copy(idx1_hbm, idx1_v)
    pltpu.sync_copy(b0_hbm, b0_v)

    one = jnp.float32(1.0)
    obufs = (ob0_v, ob1_v)
    osems = (sem_o0, sem_o1)

    def task(i, _):
        t = wid * _PER_W + i
        b = jnp.where(t >= _OUT, 1, 0)
        jz = t - b * _OUT
        kz0 = (jz * (63 * 690)) >> 16
        kz1 = jnp.minimum(kz0 + 1, _IN - 1)

        ha = pltpu.async_copy(x_hbm.at[b, kz0], pa_v, sem_ina)
        hb = pltpu.async_copy(x_hbm.at[b, kz1], pb_v, sem_inb)
        ha.wait()
        hb.wait()

        wz0 = w0_v[pl.ds(jz * _L, _L)]
        wz1 = one - wz0

        def zlerp(rr, _):
            @plsc.parallel_loop(0, _RVR, unroll=8)
            def _zb(u):
                s = pl.ds(u * _L, _L)
                pa_v[rr, s] = wz0 * pa_v[rr, s] + wz1 * pb_v[rr, s]

            return 0

        lax.fori_loop(0, _IN, zlerp, 0)

        out_handles = [None, None]
        for Q in range(4):
            ob_v = obufs[Q % 2]
            if out_handles[Q % 2] is not None:
                out_handles[Q % 2].wait()

            def yrow(jl, _):
                jy = Q * _QY + jl
                ky0 = (jy * (63 * 690)) >> 16
                ky1 = jnp.minimum(ky0 + 1, _IN - 1)
                wy0 = w0_v[pl.ds(jy * _L, _L)]
                wy1 = one - wy0
                base = jl * _ROWW

                @plsc.parallel_loop(0, _RVR, unroll=8)
                def _tyb(u):
                    s = pl.ds(u * _L, _L)
                    tyq_v[pl.ds(base + u * _L, _L)] = (
                        wy0 * pa_v[ky0, s] + wy1 * pa_v[ky1, s]
                    )

                return 0

            lax.fori_loop(0, _QY, yrow, 0)

            def xcol(v, _):
                s = pl.ds(v * _L, _L)
                iv0 = idx0_v[s]
                iv1 = idx1_v[s]
                bw0 = b0_v[s]
                bw1 = one - bw0

                @plsc.parallel_loop(0, _QY, unroll=6)
                def _xrow(jl):
                    off = jl * _ROWW
                    g0 = plsc.load_gather(tyq_v, [iv0 + off])
                    g1 = plsc.load_gather(tyq_v, [iv1 + off])
                    ob_v[jl, s] = bw0 * g0 + bw1 * g1

                return 0

            lax.fori_loop(0, _OVR, xcol, 0)

            out_handles[Q % 2] = pltpu.async_copy(
                ob_v, out_hbm.at[b, jz, pl.ds(Q * _QY, _QY)], osems[Q % 2]
            )

        out_handles[0].wait()
        out_handles[1].wait()
        return 0

    lax.fori_loop(0, _PER_W, task, 0)


@jax.jit
def kernel(x):
    B = x.shape[0]
    w0rep, idx0, idx1, b0 = _tables()
    mesh = plsc.VectorSubcoreMesh(
        core_axis_name="c", subcore_axis_name="s", num_cores=2, num_subcores=16
    )
    run = functools.partial(
        pl.kernel,
        out_type=jax.ShapeDtypeStruct((B, _OUT, _OUT, _OROWW), jnp.float32),
        mesh=mesh,
        compiler_params=pltpu.CompilerParams(needs_layout_passes=False),
        scratch_types=[
            pltpu.VMEM((_IN, _ROWW), jnp.float32),
            pltpu.VMEM((_IN, _ROWW), jnp.float32),
            pltpu.VMEM((_QY * _ROWW,), jnp.float32),
            pltpu.VMEM((_QY, _OROWW), jnp.float32),
            pltpu.VMEM((_QY, _OROWW), jnp.float32),
            pltpu.VMEM((_OUT * _L,), jnp.float32),
            pltpu.VMEM((_OROWW,), jnp.int32),
            pltpu.VMEM((_OROWW,), jnp.int32),
            pltpu.VMEM((_OROWW,), jnp.float32),
            pltpu.SemaphoreType.DMA,
            pltpu.SemaphoreType.DMA,
            pltpu.SemaphoreType.DMA,
            pltpu.SemaphoreType.DMA,
        ],
    )(_body)
    x2 = x.reshape(B, _IN, _IN, _ROWW)
    out = run(x2, jnp.asarray(w0rep), jnp.asarray(idx0), jnp.asarray(idx1),
              jnp.asarray(b0))
    return out.reshape(B, _OUT, _OUT, _OUT, _C)


# speedup vs baseline: 2.6314x; 1.0350x over previous
import functools

import jax
import jax.numpy as jnp
import numpy as np
from jax import lax
from jax.experimental import pallas as pl
from jax.experimental.pallas import tpu as pltpu
from jax.experimental.pallas import tpu_sc as plsc

_IN = 64
_OUT = 96
_C = 8
_L = 16
_ROWW = _IN * _C
_OROWW = _OUT * _C
_QY = 24
_NW = 32
_NTASK = 2 * _OUT
_PER_W = _NTASK // _NW
_RVR = _ROWW // _L
_OVR = _OROWW // _L


def _interp_1d():
    loc = np.linspace(0.0, _IN - 1.0, _OUT)
    k0 = np.clip(np.floor(loc), 0, _IN - 1).astype(np.int64)
    k1 = np.clip(k0 + 1, 0, _IN - 1)
    w0 = k1.astype(np.float64) - loc
    return k0, k1, w0.astype(np.float32)


def _tables():
    k0, k1, w0 = _interp_1d()
    w0rep = np.repeat(w0, _L).astype(np.float32)
    p = np.arange(_OUT * _C)
    jx, c = p // _C, p % _C
    idx0 = (k0[jx] * _C + c).astype(np.int32)
    idx1 = (k1[jx] * _C + c).astype(np.int32)
    b0 = w0[jx].astype(np.float32)
    return w0rep, idx0, idx1, b0


def _body(x_hbm, w0_hbm, idx0_hbm, idx1_hbm, b0_hbm, out_hbm,
          pa_v, pb_v, tyq_v, ob0_v, ob1_v, w0_v, idx0_v, idx1_v, b0_v,
          sem_ina, sem_inb, sem_o0, sem_o1):
    wid = lax.axis_index("s") * 2 + lax.axis_index("c")

    pltpu.sync_copy(w0_hbm, w0_v)
    pltpu.sync_copy(idx0_hbm, idx0_v)
    pltpu.sync_copy(idx1_hbm, idx1_v)
    pltpu.sync_copy(b0_hbm, b0_v)

    one = jnp.float32(1.0)
    obufs = (ob0_v, ob1_v)
    osems = (sem_o0, sem_o1)

    def task(i, _):
        t = wid * _PER_W + i
        b = jnp.where(t >= _OUT, 1, 0)
        jz = t - b * _OUT
        kz0 = (jz * (63 * 690)) >> 16
        kz1 = jnp.minimum(kz0 + 1, _IN - 1)

        ha = pltpu.async_copy(x_hbm.at[b, kz0], pa_v, sem_ina)
        hb = pltpu.async_copy(x_hbm.at[b, kz1], pb_v, sem_inb)
        ha.wait()
        hb.wait()

        wz0 = w0_v[pl.ds(jz * _L, _L)]
        wz1 = one - wz0

        def zlerp(rr, _):
            @plsc.parallel_loop(0, _RVR, unroll=8)
            def _zb(u):
                s = pl.ds(u * _L, _L)
                pa_v[rr, s] = wz0 * pa_v[rr, s] + wz1 * pb_v[rr, s]

            return 0

        lax.fori_loop(0, _IN, zlerp, 0)

        out_handles = [None, None]
        for Q in range(4):
            ob_v = obufs[Q % 2]
            if out_handles[Q % 2] is not None:
                out_handles[Q % 2].wait()

            def yrow(jl, _):
                jy = Q * _QY + jl
                ky0 = (jy * (63 * 690)) >> 16
                ky1 = jnp.minimum(ky0 + 1, _IN - 1)
                wy0 = w0_v[pl.ds(jy * _L, _L)]
                wy1 = one - wy0
                base = jl * _ROWW

                @plsc.parallel_loop(0, _RVR, unroll=8)
                def _tyb(u):
                    s = pl.ds(u * _L, _L)
                    tyq_v[pl.ds(base + u * _L, _L)] = (
                        wy0 * pa_v[ky0, s] + wy1 * pa_v[ky1, s]
                    )

                return 0

            lax.fori_loop(0, _QY, yrow, 0)

            def xcol(v, _):
                s = pl.ds(v * _L, _L)
                iv0 = idx0_v[s]
                iv1 = idx1_v[s]
                bw0 = b0_v[s]
                bw1 = one - bw0

                @plsc.parallel_loop(0, _QY, unroll=8)
                def _xrow(jl):
                    off = jl * _ROWW
                    g0 = plsc.load_gather(tyq_v, [iv0 + off])
                    g1 = plsc.load_gather(tyq_v, [iv1 + off])
                    ob_v[jl, s] = bw0 * g0 + bw1 * g1

                return 0

            lax.fori_loop(0, _OVR, xcol, 0)

            out_handles[Q % 2] = pltpu.async_copy(
                ob_v, out_hbm.at[b, jz, pl.ds(Q * _QY, _QY)], osems[Q % 2]
            )

        out_handles[0].wait()
        out_handles[1].wait()
        return 0

    lax.fori_loop(0, _PER_W, task, 0)


@jax.jit
def kernel(x):
    B = x.shape[0]
    w0rep, idx0, idx1, b0 = _tables()
    mesh = plsc.VectorSubcoreMesh(
        core_axis_name="c", subcore_axis_name="s", num_cores=2, num_subcores=16
    )
    run = functools.partial(
        pl.kernel,
        out_type=jax.ShapeDtypeStruct((B, _OUT, _OUT, _OROWW), jnp.float32),
        mesh=mesh,
        compiler_params=pltpu.CompilerParams(needs_layout_passes=False),
        scratch_types=[
            pltpu.VMEM((_IN, _ROWW), jnp.float32),
            pltpu.VMEM((_IN, _ROWW), jnp.float32),
            pltpu.VMEM((_QY * _ROWW,), jnp.float32),
            pltpu.VMEM((_QY, _OROWW), jnp.float32),
            pltpu.VMEM((_QY, _OROWW), jnp.float32),
            pltpu.VMEM((_OUT * _L,), jnp.float32),
            pltpu.VMEM((_OROWW,), jnp.int32),
            pltpu.VMEM((_OROWW,), jnp.int32),
            pltpu.VMEM((_OROWW,), jnp.float32),
            pltpu.SemaphoreType.DMA,
            pltpu.SemaphoreType.DMA,
            pltpu.SemaphoreType.DMA,
            pltpu.SemaphoreType.DMA,
        ],
    )(_body)
    x2 = x.reshape(B, _IN, _IN, _ROWW)
    out = run(x2, jnp.asarray(w0rep), jnp.asarray(idx0), jnp.asarray(idx1),
              jnp.asarray(b0))
    return out.reshape(B, _OUT, _OUT, _OUT, _C)
